# Initial kernel scaffold; baseline (speedup 1.0000x reference)
import jax
import jax.numpy as jnp
from jax.experimental import pallas as pl


def kernel(x, r_w1, r_b1, ln_scale, ln_bias, r_w2, r_b2, ew1, eb1, ew2, eb2, expert_priors):
    raise NotImplementedError("write your pallas kernel here")


# speedup vs baseline: 1.8582x; 1.8582x over previous
import functools

import jax
import jax.numpy as jnp
from jax.experimental import pallas as pl
from jax.experimental.pallas import tpu as pltpu

_EMBED_DIM = 768
_FF_DIM = 3072
_NUM_EXPERTS = 8
_TOP_K = 2
_F_BLK = 512
_NF = _FF_DIM // _F_BLK


def _gelu(x):
    sqrt_2_pi = 0.7978845608028654
    coef = 0.044715
    x3 = x ** 3
    inner = sqrt_2_pi * (x + coef * x3)
    return 0.5 * x * (1.0 + jnp.tanh(inner))


def _router_kernel(x_ref, w1_ref, b1_ref, lns_ref, lnb_ref, w2_ref, b2_ref,
                   mask_ref):
    x = x_ref[...]
    h = jnp.dot(x, w1_ref[...], preferred_element_type=jnp.float32)
    h = h + b1_ref[...]
    mean = jnp.mean(h, axis=-1, keepdims=True)
    var = jnp.mean(jnp.square(h - mean), axis=-1, keepdims=True)
    h = (h - mean) * jax.lax.rsqrt(var + 1e-6) * lns_ref[...] + lnb_ref[...]
    h = _gelu(h)
    logits = jnp.dot(h, w2_ref[...], preferred_element_type=jnp.float32)
    logits = logits + b2_ref[...]
    lmax = jnp.max(logits, axis=-1, keepdims=True)
    ex = jnp.exp(logits - lmax)
    p = ex / jnp.sum(ex, axis=-1, keepdims=True)
    s, e = p.shape
    iota = jax.lax.broadcasted_iota(jnp.int32, (s, e), 1)
    m1 = jnp.max(p, axis=-1, keepdims=True)
    i1 = jnp.min(jnp.where(p == m1, iota, e), axis=-1, keepdims=True)
    oh1 = iota == i1
    p2 = jnp.where(oh1, -jnp.inf, p)
    m2 = jnp.max(p2, axis=-1, keepdims=True)
    i2 = jnp.min(jnp.where(p2 == m2, iota, e), axis=-1, keepdims=True)
    oh2 = iota == i2
    denom = m1 + m2
    mask_ref[...] = (jnp.where(oh1, m1, 0.0) + jnp.where(oh2, m2, 0.0)) / denom


def _expert_kernel(mask_ref, x_ref, w1a_ref, w1b_ref, b1a_ref, b1b_ref,
                   w2_ref, b2_ref, out_ref):
    e = pl.program_id(0)
    j = pl.program_id(1)

    @pl.when(jnp.logical_and(e == 0, j == 0))
    def _init():
        out_ref[...] = jnp.zeros_like(out_ref)

    x = x_ref[...]
    h1 = jnp.dot(x, w1a_ref[0], preferred_element_type=jnp.float32)
    h1 = h1 + b1a_ref[0]
    h2 = jnp.dot(x, w1b_ref[0], preferred_element_type=jnp.float32)
    h2 = h2 + b1b_ref[0]
    g = h1 * _gelu(h2)

    m = mask_ref[...]
    s, ne = m.shape
    iota = jax.lax.broadcasted_iota(jnp.int32, (s, ne), 1)
    col = jnp.sum(jnp.where(iota == e, m, 0.0), axis=-1, keepdims=True)

    out_ref[...] += jnp.dot(g * col, w2_ref[0],
                            preferred_element_type=jnp.float32)

    @pl.when(j == _NF - 1)
    def _bias():
        out_ref[...] += col * b2_ref[0]


@jax.jit
def kernel(x, r_w1, r_b1, ln_scale, ln_bias, r_w2, r_b2, ew1, eb1, ew2, eb2,
           expert_priors):
    del expert_priors
    b, s, d = x.shape
    f = _FF_DIM
    e = _NUM_EXPERTS
    x2d = x.reshape(s, d)

    mask = pl.pallas_call(
        _router_kernel,
        out_shape=jax.ShapeDtypeStruct((s, e), jnp.float32),
    )(x2d, r_w1, r_b1.reshape(1, -1), ln_scale.reshape(1, -1),
      ln_bias.reshape(1, -1), r_w2, r_b2.reshape(1, -1))

    eb1_3d = eb1.reshape(e, 1, 2 * f)
    eb2_3d = eb2.reshape(e, 1, d)

    out = pl.pallas_call(
        _expert_kernel,
        grid=(e, _NF),
        in_specs=[
            pl.BlockSpec((s, e), lambda ei, j: (0, 0)),
            pl.BlockSpec((s, d), lambda ei, j: (0, 0)),
            pl.BlockSpec((1, d, _F_BLK), lambda ei, j: (ei, 0, j)),
            pl.BlockSpec((1, d, _F_BLK), lambda ei, j: (ei, 0, j + _NF)),
            pl.BlockSpec((1, 1, _F_BLK), lambda ei, j: (ei, 0, j)),
            pl.BlockSpec((1, 1, _F_BLK), lambda ei, j: (ei, 0, j + _NF)),
            pl.BlockSpec((1, _F_BLK, d), lambda ei, j: (ei, j, 0)),
            pl.BlockSpec((1, 1, d), lambda ei, j: (ei, 0, 0)),
        ],
        out_specs=pl.BlockSpec((s, d), lambda ei, j: (0, 0)),
        out_shape=jax.ShapeDtypeStruct((s, d), jnp.float32),
        compiler_params=pltpu.CompilerParams(
            dimension_semantics=("arbitrary", "arbitrary"),
        ),
    )(mask, x2d, ew1, ew1, eb1_3d, eb1_3d, ew2, eb2_3d)

    return (out.reshape(b, s, d), 0.0)
